# Initial kernel scaffold; baseline (speedup 1.0000x reference)
import functools

import jax
import jax.numpy as jnp
from jax import lax
from jax.experimental import pallas as pl
from jax.experimental.pallas import tpu as pltpu
from jax.experimental.pallas import tpu_sc as plsc

_NUM_WORKERS = 32
_CHUNK = 128


def _emb_body(n_chunks, per_worker, idx_hbm, table_hbm, out_hbm,
              idx_v, rows_v, gsem):
    wid = lax.axis_index("s") * 2 + lax.axis_index("c")
    pltpu.sync_copy(idx_hbm.at[wid], idx_v)
    base = wid * per_worker

    @pl.loop(0, n_chunks)
    def _(j):
        pltpu.async_copy(table_hbm.at[idx_v.at[j]], rows_v, gsem).wait()
        pltpu.sync_copy(rows_v, out_hbm.at[pl.ds(base + j * _CHUNK, _CHUNK)])


def kernel(inputs, embeddings):
    b, s = inputs.shape
    v, d = embeddings.shape
    total = b * s
    per_worker = total // _NUM_WORKERS
    n_chunks = per_worker // _CHUNK
    idx = inputs.astype(jnp.int32).reshape(_NUM_WORKERS, n_chunks, _CHUNK)

    mesh = plsc.VectorSubcoreMesh(core_axis_name="c", subcore_axis_name="s")
    emb = pl.kernel(
        functools.partial(_emb_body, n_chunks, per_worker),
        out_type=jax.ShapeDtypeStruct((total, d), jnp.float32),
        mesh=mesh,
        scratch_types=[
            pltpu.VMEM((n_chunks, _CHUNK), jnp.int32),
            pltpu.VMEM((_CHUNK, d), jnp.float32),
            pltpu.SemaphoreType.DMA,
        ],
    )
    out = emb(idx, embeddings)
    return out.reshape(b, s, d)


# speedup vs baseline: 1.0220x; 1.0220x over previous
import functools

import jax
import jax.numpy as jnp
from jax import lax
from jax.experimental import pallas as pl
from jax.experimental.pallas import tpu as pltpu
from jax.experimental.pallas import tpu_sc as plsc

_NUM_WORKERS = 32
_CHUNK = 128


def _emb_body(n_chunks, per_worker, idx_hbm, table_hbm, out_hbm,
              idx_v, rows_v, gsem):
    wid = lax.axis_index("s") * 2 + lax.axis_index("c")
    pltpu.sync_copy(idx_hbm.at[wid], idx_v)
    base = wid * per_worker

    @pl.loop(0, n_chunks)
    def _(j):
        pltpu.async_copy(table_hbm.at[idx_v.at[j]], rows_v, gsem).wait()
        pltpu.sync_copy(rows_v, out_hbm.at[pl.ds(base + j * _CHUNK, _CHUNK)])


def kernel(inputs, embeddings):
    b, s = inputs.shape
    v, d = embeddings.shape
    total = b * s
    per_worker = total // _NUM_WORKERS
    n_chunks = per_worker // _CHUNK
    idx = inputs.astype(jnp.int32).reshape(_NUM_WORKERS, n_chunks, _CHUNK)

    mesh = plsc.VectorSubcoreMesh(core_axis_name="c", subcore_axis_name="s")
    emb = pl.kernel(
        functools.partial(_emb_body, n_chunks, per_worker),
        out_type=jax.ShapeDtypeStruct((total, d), jnp.float32),
        mesh=mesh,
        scratch_types=[
            pltpu.VMEM((n_chunks, _CHUNK), jnp.int32),
            pltpu.VMEM((_CHUNK, d), jnp.float32),
            pltpu.SemaphoreType.DMA,
        ],
        compiler_params=pltpu.CompilerParams(use_tc_tiling_on_sc=False),
    )
    out = emb(idx, embeddings)
    return out.reshape(b, s, d)


# speedup vs baseline: 1.1106x; 1.0867x over previous
import functools

import jax
import jax.numpy as jnp
from jax import lax
from jax.experimental import pallas as pl
from jax.experimental.pallas import tpu as pltpu
from jax.experimental.pallas import tpu_sc as plsc

_NUM_WORKERS = 32
_CHUNK = 128
_GRP = 4


def _emb_body(n_groups, per_worker, idx_hbm, table_hbm, out_hbm,
              idx_v, buf_a, buf_b, gsem_a, gsem_b, ssem_a, ssem_b):
    wid = lax.axis_index("s") * 2 + lax.axis_index("c")
    pltpu.sync_copy(idx_hbm.at[wid], idx_v)
    base = wid * per_worker
    rows_per_grp = _GRP * _CHUNK

    def fire_gathers(g, buf, sem):
        for b in range(_GRP):
            pltpu.async_copy(
                table_hbm.at[idx_v.at[g * _GRP + b]],
                buf.at[pl.ds(b * _CHUNK, _CHUNK)], sem)

    def drain_gathers(buf, sem):
        pltpu.make_async_copy(
            table_hbm.at[pl.ds(0, rows_per_grp)], buf, sem).wait()

    def store_rows(g, buf, sem):
        return pltpu.async_copy(
            buf, out_hbm.at[pl.ds(base + g * rows_per_grp, rows_per_grp)], sem)

    fire_gathers(0, buf_a, gsem_a)

    @pl.loop(0, n_groups, step=2)
    def _(gi):
        fire_gathers(gi + 1, buf_b, gsem_b)
        drain_gathers(buf_a, gsem_a)
        store_rows(gi, buf_a, ssem_a).wait()

        @pl.when(gi + 2 < n_groups)
        def _():
            fire_gathers(gi + 2, buf_a, gsem_a)

        drain_gathers(buf_b, gsem_b)
        store_rows(gi + 1, buf_b, ssem_b).wait()


def kernel(inputs, embeddings):
    b, s = inputs.shape
    v, d = embeddings.shape
    total = b * s
    per_worker = total // _NUM_WORKERS
    n_chunks = per_worker // _CHUNK
    n_groups = n_chunks // _GRP
    idx = inputs.astype(jnp.int32).reshape(_NUM_WORKERS, n_chunks, _CHUNK)

    mesh = plsc.VectorSubcoreMesh(core_axis_name="c", subcore_axis_name="s")
    emb = pl.kernel(
        functools.partial(_emb_body, n_groups, per_worker),
        out_type=jax.ShapeDtypeStruct((total, d), jnp.float32),
        mesh=mesh,
        scratch_types=[
            pltpu.VMEM((n_chunks, _CHUNK), jnp.int32),
            pltpu.VMEM((_GRP * _CHUNK, d), jnp.float32),
            pltpu.VMEM((_GRP * _CHUNK, d), jnp.float32),
            pltpu.SemaphoreType.DMA,
            pltpu.SemaphoreType.DMA,
            pltpu.SemaphoreType.DMA,
            pltpu.SemaphoreType.DMA,
        ],
        compiler_params=pltpu.CompilerParams(use_tc_tiling_on_sc=False),
    )
    out = emb(idx, embeddings)
    return out.reshape(b, s, d)


# speedup vs baseline: 1.7984x; 1.6192x over previous
import functools

import jax
import jax.numpy as jnp
from jax import lax
from jax.experimental import pallas as pl
from jax.experimental.pallas import tpu as pltpu
from jax.experimental.pallas import tpu_sc as plsc

_NUM_WORKERS = 32
_GRP = 8


def _emb_body(n_groups, rows_per_worker, idx_hbm, table_hbm, out_hbm,
              idx_v, buf_a, buf_b, gsem_a, gsem_b, ssem_a, ssem_b):
    wid = lax.axis_index("s") * 2 + lax.axis_index("c")
    base = wid * rows_per_worker
    pltpu.sync_copy(idx_hbm.at[pl.ds(base, rows_per_worker)], idx_v)

    def fire_gathers(g, buf, sem):
        for r in range(_GRP):
            pltpu.async_copy(
                table_hbm.at[idx_v.at[g * _GRP + r]],
                buf.at[r], sem)

    def drain_gathers(buf, sem):
        pltpu.make_async_copy(
            out_hbm.at[pl.ds(0, _GRP)], buf, sem).wait()

    def store_rows(g, buf, sem):
        return pltpu.async_copy(
            buf, out_hbm.at[pl.ds(base + g * _GRP, _GRP)], sem)

    fire_gathers(0, buf_a, gsem_a)

    @pl.loop(0, n_groups, step=2)
    def _(gi):
        fire_gathers(gi + 1, buf_b, gsem_b)
        drain_gathers(buf_a, gsem_a)
        store_rows(gi, buf_a, ssem_a).wait()

        @pl.when(gi + 2 < n_groups)
        def _():
            fire_gathers(gi + 2, buf_a, gsem_a)

        drain_gathers(buf_b, gsem_b)
        store_rows(gi + 1, buf_b, ssem_b).wait()


def kernel(inputs, embeddings):
    b, s = inputs.shape
    v, d = embeddings.shape
    rows_per_worker = b // _NUM_WORKERS
    n_groups = rows_per_worker // _GRP
    idx = inputs.astype(jnp.int32)

    mesh = plsc.VectorSubcoreMesh(core_axis_name="c", subcore_axis_name="s")
    emb = pl.kernel(
        functools.partial(_emb_body, n_groups, rows_per_worker),
        out_type=jax.ShapeDtypeStruct((b, s, d), jnp.float32),
        mesh=mesh,
        scratch_types=[
            pltpu.VMEM((rows_per_worker, s), jnp.int32),
            pltpu.VMEM((_GRP, s, d), jnp.float32),
            pltpu.VMEM((_GRP, s, d), jnp.float32),
            pltpu.SemaphoreType.DMA,
            pltpu.SemaphoreType.DMA,
            pltpu.SemaphoreType.DMA,
            pltpu.SemaphoreType.DMA,
        ],
        compiler_params=pltpu.CompilerParams(use_tc_tiling_on_sc=False),
    )
    return emb(idx, embeddings)
